# baseline (speedup 1.0000x reference)
import jax
import jax.numpy as jnp
import numpy as np
from jax.experimental import pallas as pl
from jax.experimental.pallas import tpu as pltpu

_G = 152
_NA = 3
_NF = 10
_R = 8
_ANCHOR_W = (1.08, 3.42, 6.63)
_ANCHOR_H = (1.19, 4.41, 11.38)


def _decode_block(stride_ref, x_ref, o_ref):
    a = pl.program_id(1)
    rchunk = pl.program_id(2)
    s = stride_ref[0, 0]

    aw = jnp.where(a == 0, _ANCHOR_W[0], jnp.where(a == 1, _ANCHOR_W[1], _ANCHOR_W[2]))
    ah = jnp.where(a == 0, _ANCHOR_H[0], jnp.where(a == 1, _ANCHOR_H[1], _ANCHOR_H[2]))

    v = x_ref[0, 0]
    jj = jax.lax.broadcasted_iota(jnp.int32, (_R, _G), 1).astype(jnp.float32)
    ii = (jax.lax.broadcasted_iota(jnp.int32, (_R, _G), 0) + rchunk * _R).astype(jnp.float32)

    sig = jax.nn.sigmoid
    planes = [
        (sig(v[0]) + jj) * s,
        (sig(v[1]) + ii) * s,
        jnp.exp(v[2]) * aw,
        jnp.exp(v[3]) * ah,
        v[4],
        v[5],
        sig(v[6]),
        sig(v[7]),
        sig(v[8]),
        sig(v[9]),
    ]
    dec = jnp.stack(planes, axis=0)
    o_ref[0, 0] = jnp.transpose(dec, (1, 2, 0))


def kernel(x, img_size):
    n = x.shape[0]
    x5 = x.reshape(n, _NA, _NF, _G, _G)
    stride = (jnp.float32(img_size) / _G).reshape(1, 1)

    out = pl.pallas_call(
        _decode_block,
        grid=(n, _NA, _G // _R),
        in_specs=[
            pl.BlockSpec(memory_space=pltpu.SMEM),
            pl.BlockSpec((1, 1, _NF, _R, _G), lambda b, a, r: (b, a, 0, r, 0)),
        ],
        out_specs=pl.BlockSpec((1, 1, _R, _G, _NF), lambda b, a, r: (b, a, r, 0, 0)),
        out_shape=jax.ShapeDtypeStruct((n, _NA, _G, _G, _NF), jnp.float32),
    )(stride, x5)
    return out.reshape(n, _NA * _G * _G, _NF)


# speedup vs baseline: 2.5112x; 2.5112x over previous
import jax
import jax.numpy as jnp
import numpy as np
from jax.experimental import pallas as pl
from jax.experimental.pallas import tpu as pltpu

_G = 152
_NA = 3
_NF = 10
_R = 152
_ANCHOR_W = (1.08, 3.42, 6.63)
_ANCHOR_H = (1.19, 4.41, 11.38)


def _decode_block(stride_ref, x_ref, o_ref):
    a = pl.program_id(1)
    rchunk = pl.program_id(2)
    s = stride_ref[0, 0]

    aw = jnp.where(a == 0, _ANCHOR_W[0], jnp.where(a == 1, _ANCHOR_W[1], _ANCHOR_W[2]))
    ah = jnp.where(a == 0, _ANCHOR_H[0], jnp.where(a == 1, _ANCHOR_H[1], _ANCHOR_H[2]))

    v = x_ref[0, 0]
    jj = jax.lax.broadcasted_iota(jnp.int32, (_R, _G), 1).astype(jnp.float32)
    ii = (jax.lax.broadcasted_iota(jnp.int32, (_R, _G), 0) + rchunk * _R).astype(jnp.float32)

    sig = jax.nn.sigmoid
    planes = [
        (sig(v[0]) + jj) * s,
        (sig(v[1]) + ii) * s,
        jnp.exp(v[2]) * aw,
        jnp.exp(v[3]) * ah,
        v[4],
        v[5],
        sig(v[6]),
        sig(v[7]),
        sig(v[8]),
        sig(v[9]),
    ]
    dec = jnp.stack(planes, axis=0)
    o_ref[0, 0] = jnp.transpose(dec, (1, 2, 0))


def kernel(x, img_size):
    n = x.shape[0]
    x5 = x.reshape(n, _NA, _NF, _G, _G)
    stride = (jnp.float32(img_size) / _G).reshape(1, 1)

    out = pl.pallas_call(
        _decode_block,
        grid=(n, _NA, _G // _R),
        in_specs=[
            pl.BlockSpec(memory_space=pltpu.SMEM),
            pl.BlockSpec((1, 1, _NF, _R, _G), lambda b, a, r: (b, a, 0, r, 0)),
        ],
        out_specs=pl.BlockSpec((1, 1, _R, _G, _NF), lambda b, a, r: (b, a, r, 0, 0)),
        out_shape=jax.ShapeDtypeStruct((n, _NA, _G, _G, _NF), jnp.float32),
    )(stride, x5)
    return out.reshape(n, _NA * _G * _G, _NF)
